# baseline (speedup 1.0000x reference)
import functools

import jax
import jax.numpy as jnp
from jax import lax
from jax.experimental import pallas as pl
from jax.experimental.pallas import tpu as pltpu
from jax.experimental.pallas import tpu_sc as plsc

NUM_NODES = 1000000
EMBED_DIM = 64
BATCH = 16384

NC = 2
NS = 16
LANES = 16
NW = NC * NS
BPW = BATCH // NW
CHUNK = 128
NCHUNK = BPW // CHUNK
GROUPS = BPW // LANES

_mesh = plsc.VectorSubcoreMesh(
    core_axis_name="c", subcore_axis_name="s", num_cores=NC, num_subcores=NS
)


@functools.partial(
    pl.kernel,
    out_type=jax.ShapeDtypeStruct((BATCH,), jnp.float32),
    mesh=_mesh,
    scratch_types=[
        pltpu.VMEM((NCHUNK, CHUNK), jnp.int32),
        pltpu.VMEM((NCHUNK, CHUNK), jnp.int32),
        pltpu.VMEM((BPW, EMBED_DIM), jnp.float32),
        pltpu.VMEM((BPW, EMBED_DIM), jnp.float32),
        pltpu.VMEM((BPW,), jnp.float32),
        pltpu.SemaphoreType.DMA,
    ],
    compiler_params=pltpu.CompilerParams(
        needs_layout_passes=False, use_tc_tiling_on_sc=False),
)
def _score_kernel(sidx_hbm, oidx_hbm, emb_hbm, out_hbm,
                  sidx_v, oidx_v, srows_v, orows_v, out_v, sem):
    wid = lax.axis_index("s") * NC + lax.axis_index("c")
    base = wid * NCHUNK

    pltpu.sync_copy(sidx_hbm.at[pl.ds(base, NCHUNK)], sidx_v)
    pltpu.sync_copy(oidx_hbm.at[pl.ds(base, NCHUNK)], oidx_v)

    copies = []
    for j in range(NCHUNK):
        copies.append(pltpu.async_copy(
            emb_hbm.at[sidx_v.at[j]],
            srows_v.at[pl.ds(j * CHUNK, CHUNK)], sem))
        copies.append(pltpu.async_copy(
            emb_hbm.at[oidx_v.at[j]],
            orows_v.at[pl.ds(j * CHUNK, CHUNK)], sem))
    for c in copies:
        c.wait()

    lane = jnp.arange(LANES, dtype=jnp.int32)

    def group_body(g, carry):
        rows = g * LANES + lane
        acc = jnp.zeros((LANES,), jnp.float32)
        for d in range(EMBED_DIM):
            col = jnp.full((LANES,), d, jnp.int32)
            sv = plsc.load_gather(srows_v, [rows, col])
            ov = plsc.load_gather(orows_v, [rows, col])
            acc = acc + sv * ov
        out_v[pl.ds(g * LANES, LANES)] = acc
        return carry

    lax.fori_loop(0, GROUPS, group_body, 0)

    pltpu.sync_copy(out_v, out_hbm.at[pl.ds(wid * BPW, BPW)])


def kernel(triples, entity_embedding):
    sidx = triples[:, 0].reshape(NW * NCHUNK, CHUNK)
    oidx = triples[:, 1].reshape(NW * NCHUNK, CHUNK)
    return _score_kernel(sidx, oidx, entity_embedding)
